# baseline (speedup 1.0000x reference)
import functools

import jax
import jax.numpy as jnp
from jax import lax
from jax.experimental import pallas as pl
from jax.experimental.pallas import tpu as pltpu
from jax.experimental.pallas import tpu_sc as plsc

_NC = 2
_NS = 16
_CH = 128


def _sc_mesh():
    return plsc.VectorSubcoreMesh(core_axis_name="c", subcore_axis_name="s")


def _sc_hist(col2d, zeros_pad):
    nch_all, _ = col2d.shape
    npad = zeros_pad.shape[0]
    nw = _NC * _NS
    nch = nch_all // nw
    rpt = npad // _NS

    @functools.partial(
        pl.kernel,
        out_type=jax.ShapeDtypeStruct((_NC, npad), jnp.float32),
        mesh=_sc_mesh(),
        scratch_types=[
            pltpu.VMEM_SHARED((npad,), jnp.float32),
            pltpu.VMEM((nch, _CH), jnp.int32),
            pltpu.VMEM((_CH,), jnp.float32),
        ],
    )
    def k(col_hbm, z_hbm, out_hbm, acc, ic_all, ones_v):
        cid = lax.axis_index("c")
        sid = lax.axis_index("s")
        wid = sid * _NC + cid
        pltpu.sync_copy(z_hbm.at[pl.ds(sid * rpt, rpt)],
                        acc.at[pl.ds(sid * rpt, rpt)])
        pltpu.sync_copy(col_hbm.at[pl.ds(wid * nch, nch)], ic_all)
        for j in range(_CH // 16):
            ones_v[pl.ds(j * 16, 16)] = jnp.ones((16,), jnp.float32)
        plsc.subcore_barrier()

        def body(ch, carry):
            pltpu.sync_copy(ones_v, acc.at[ic_all.at[ch]], add=True)
            return carry

        lax.fori_loop(0, nch, body, 0)
        plsc.subcore_barrier()
        pltpu.sync_copy(acc.at[pl.ds(sid * rpt, rpt)],
                        out_hbm.at[cid].at[pl.ds(sid * rpt, rpt)])

    return k(col2d, zeros_pad)


def _sc_row_scatter(xws, row2d, col2d):
    n, d = xws.shape
    nch_all, _ = row2d.shape
    nw = _NC * _NS
    nch = nch_all // nw
    rpt = n // _NS

    @functools.partial(
        pl.kernel,
        out_type=jax.ShapeDtypeStruct((_NC, n, d), jnp.float32),
        mesh=_sc_mesh(),
        scratch_types=[
            pltpu.VMEM_SHARED((n, d), jnp.float32),
            pltpu.VMEM((nch, _CH), jnp.int32),
            pltpu.VMEM((_CH,), jnp.int32),
            pltpu.VMEM((_CH,), jnp.int32),
            pltpu.VMEM((_CH, d), jnp.float32),
            pltpu.VMEM((_CH, d), jnp.float32),
            pltpu.SemaphoreType.DMA,
            pltpu.SemaphoreType.DMA,
            pltpu.SemaphoreType.DMA,
            pltpu.SemaphoreType.DMA,
        ],
    )
    def k(xws_hbm, row_hbm, col_hbm, out_hbm, acc, ir_all, ic0, ic1,
          rows0, rows1, semg0, semg1, sems0, sems1):
        cid = lax.axis_index("c")
        sid = lax.axis_index("s")
        wid = sid * _NC + cid
        pltpu.sync_copy(xws_hbm.at[pl.ds(sid * rpt, rpt)],
                        acc.at[pl.ds(sid * rpt, rpt)])
        pltpu.sync_copy(row_hbm.at[pl.ds(wid * nch, nch)], ir_all)
        plsc.subcore_barrier()

        pltpu.async_copy(xws_hbm.at[ir_all.at[0]], rows0, semg0)
        pltpu.sync_copy(col_hbm.at[wid * nch], ic0)
        pltpu.async_copy(xws_hbm.at[ir_all.at[1]], rows1, semg1)
        pltpu.sync_copy(col_hbm.at[wid * nch + 1], ic1)

        def body(it, carry):
            ch0 = it * 2
            ch1 = ch0 + 1
            pltpu.make_async_copy(xws_hbm.at[ir_all.at[ch0]], rows0,
                                  semg0).wait()
            pltpu.async_copy(rows0, acc.at[ic0], sems0, add=True)
            pltpu.make_async_copy(xws_hbm.at[ir_all.at[ch1]], rows1,
                                  semg1).wait()
            pltpu.async_copy(rows1, acc.at[ic1], sems1, add=True)

            @pl.when(ch0 + 2 < nch)
            def _():
                pltpu.make_async_copy(rows0, acc.at[ic0], sems0).wait()
                pltpu.async_copy(xws_hbm.at[ir_all.at[ch0 + 2]], rows0, semg0)
                pltpu.sync_copy(col_hbm.at[wid * nch + ch0 + 2], ic0)
                pltpu.make_async_copy(rows1, acc.at[ic1], sems1).wait()
                pltpu.async_copy(xws_hbm.at[ir_all.at[ch1 + 2]], rows1, semg1)
                pltpu.sync_copy(col_hbm.at[wid * nch + ch1 + 2], ic1)

            return carry

        lax.fori_loop(0, nch // 2, body, 0)
        pltpu.make_async_copy(rows0, acc.at[ic0], sems0).wait()
        pltpu.make_async_copy(rows1, acc.at[ic1], sems1).wait()
        plsc.subcore_barrier()
        pltpu.sync_copy(acc.at[pl.ds(sid * rpt, rpt)],
                        out_hbm.at[cid].at[pl.ds(sid * rpt, rpt)])

    return k(xws, row2d, col2d)


def _sc_scalar_scatter(s1, row2d, col2d, zeros_pad):
    npad = zeros_pad.shape[0]
    nch_all, _ = row2d.shape
    nw = _NC * _NS
    nch = nch_all // nw
    rpt = npad // _NS

    @functools.partial(
        pl.kernel,
        out_type=jax.ShapeDtypeStruct((_NC, npad), jnp.float32),
        mesh=_sc_mesh(),
        scratch_types=[
            pltpu.VMEM_SHARED((npad,), jnp.float32),
            pltpu.VMEM((nch, _CH), jnp.int32),
            pltpu.VMEM((nch, _CH), jnp.int32),
            pltpu.VMEM((_CH,), jnp.float32),
            pltpu.VMEM((_CH,), jnp.float32),
            pltpu.SemaphoreType.DMA,
            pltpu.SemaphoreType.DMA,
        ],
    )
    def k(s1_hbm, row_hbm, col_hbm, z_hbm, out_hbm, acc, ir_all, ic_all,
          val0, val1, sem0, sem1):
        cid = lax.axis_index("c")
        sid = lax.axis_index("s")
        wid = sid * _NC + cid
        pltpu.sync_copy(z_hbm.at[pl.ds(sid * rpt, rpt)],
                        acc.at[pl.ds(sid * rpt, rpt)])
        pltpu.sync_copy(row_hbm.at[pl.ds(wid * nch, nch)], ir_all)
        pltpu.sync_copy(col_hbm.at[pl.ds(wid * nch, nch)], ic_all)
        plsc.subcore_barrier()

        pltpu.async_copy(s1_hbm.at[ir_all.at[0]], val0, sem0)

        def body(it, carry):
            ch0 = it * 2
            ch1 = ch0 + 1
            pltpu.async_copy(s1_hbm.at[ir_all.at[ch1]], val1, sem1)
            pltpu.make_async_copy(s1_hbm.at[ir_all.at[ch0]], val0,
                                  sem0).wait()
            pltpu.sync_copy(val0, acc.at[ic_all.at[ch0]], add=True)

            @pl.when(ch0 + 2 < nch)
            def _():
                pltpu.async_copy(s1_hbm.at[ir_all.at[ch0 + 2]], val0, sem0)

            pltpu.make_async_copy(s1_hbm.at[ir_all.at[ch1]], val1,
                                  sem1).wait()
            pltpu.sync_copy(val1, acc.at[ic_all.at[ch1]], add=True)
            return carry

        lax.fori_loop(0, nch // 2, body, 0)
        plsc.subcore_barrier()
        pltpu.sync_copy(acc.at[pl.ds(sid * rpt, rpt)],
                        out_hbm.at[cid].at[pl.ds(sid * rpt, rpt)])

    return k(s1, row2d, col2d, zeros_pad)


_BLK = 400


def _tc_matmul(x, w):
    n, din = x.shape
    hid = w.shape[1]
    grid = n // _BLK

    def body(x_ref, w_ref, xw_ref):
        xw_ref[...] = jnp.dot(x_ref[...], w_ref[...],
                              preferred_element_type=jnp.float32)

    return pl.pallas_call(
        body,
        grid=(grid,),
        in_specs=[
            pl.BlockSpec((_BLK, din), lambda i: (i, 0)),
            pl.BlockSpec((din, hid), lambda i: (0, 0)),
        ],
        out_specs=pl.BlockSpec((_BLK, hid), lambda i: (i, 0)),
        out_shape=jax.ShapeDtypeStruct((n, hid), jnp.float32),
    )(x, w)


def _tc_scale(xw, d0, d1):
    n, hid = xw.shape
    grid = n // _BLK

    def body(xw_ref, d0_ref, d1_ref, xws_ref, dinv_ref):
        dinv = lax.rsqrt(d0_ref[...] + d1_ref[...] + 1.0)
        xws_ref[...] = xw_ref[...] * dinv
        dinv_ref[...] = dinv

    return pl.pallas_call(
        body,
        grid=(grid,),
        in_specs=[
            pl.BlockSpec((_BLK, hid), lambda i: (i, 0)),
            pl.BlockSpec((_BLK, 1), lambda i: (i, 0)),
            pl.BlockSpec((_BLK, 1), lambda i: (i, 0)),
        ],
        out_specs=[
            pl.BlockSpec((_BLK, hid), lambda i: (i, 0)),
            pl.BlockSpec((_BLK, 1), lambda i: (i, 0)),
        ],
        out_shape=[
            jax.ShapeDtypeStruct((n, hid), jnp.float32),
            jax.ShapeDtypeStruct((n, 1), jnp.float32),
        ],
    )(xw, d0, d1)


def _tc_finish_conv(p0, p1, xws, dinv, b, w2):
    n, hid = p0.shape
    grid = n // _BLK

    def body(p0_ref, p1_ref, xws_ref, dinv_ref, b_ref, w2_ref, h_ref, s_ref):
        agg = p0_ref[...] + p1_ref[...] - xws_ref[...]
        h = jnp.maximum(dinv_ref[...] * agg + b_ref[...], 0.0)
        h_ref[...] = h
        s_ref[...] = jnp.dot(h, w2_ref[...], preferred_element_type=jnp.float32)

    return pl.pallas_call(
        body,
        grid=(grid,),
        in_specs=[
            pl.BlockSpec((_BLK, hid), lambda i: (i, 0)),
            pl.BlockSpec((_BLK, hid), lambda i: (i, 0)),
            pl.BlockSpec((_BLK, hid), lambda i: (i, 0)),
            pl.BlockSpec((_BLK, 1), lambda i: (i, 0)),
            pl.BlockSpec((1, hid), lambda i: (0, 0)),
            pl.BlockSpec((hid, 2), lambda i: (0, 0)),
        ],
        out_specs=[
            pl.BlockSpec((_BLK, hid), lambda i: (i, 0)),
            pl.BlockSpec((_BLK, 2), lambda i: (i, 0)),
        ],
        out_shape=[
            jax.ShapeDtypeStruct((n, hid), jnp.float32),
            jax.ShapeDtypeStruct((n, 2), jnp.float32),
        ],
    )(p0, p1, xws, dinv, b, w2)


def _tc_counts(batch_row, num_graphs, ratio):
    n = batch_row.shape[1]

    def body(b_ref, k_ref, kf_ref):
        g = lax.broadcasted_iota(jnp.int32, (num_graphs, n), 0)
        eq = (b_ref[...] == g).astype(jnp.float32)
        counts = jnp.sum(eq, axis=1, keepdims=True)
        kk = jnp.ceil(ratio * counts)
        k_ref[...] = kk
        kf_ref[...] = jnp.maximum(kk, 1.0)

    return pl.pallas_call(
        body,
        out_shape=[
            jax.ShapeDtypeStruct((num_graphs, 1), jnp.float32),
            jax.ShapeDtypeStruct((num_graphs, 1), jnp.float32),
        ],
    )(batch_row)


_RBLK = 200


def _tc_rank_gate(sa0, sa1, sroot, brel, sa0r, sa1r, srootr, bat_c, bat_r, karr):
    n = sa0.shape[0]
    ng = karr.shape[0]
    grid = n // _RBLK

    def body(sa0_ref, sa1_ref, sroot_ref, brel_ref, sa0r_ref, sa1r_ref,
             srootr_ref, batc_ref, batr_ref, k_ref, gate_ref):
        i = pl.program_id(0)
        si = sa0_ref[...] + sa1_ref[...] + brel_ref[...] + sroot_ref[...]
        sj = sa0r_ref[...] + sa1r_ref[...] + brel_ref[...] + srootr_ref[...]
        bi = batc_ref[...]
        bj = batr_ref[...]
        pos_i = i * _RBLK + lax.broadcasted_iota(jnp.int32, (_RBLK, 1), 0)
        pos_j = lax.broadcasted_iota(jnp.int32, (1, n), 1)
        beats = (sj > si) | ((sj == si) & (pos_j < pos_i))
        cmp = ((bj == bi) & beats).astype(jnp.float32)
        rank = jnp.sum(cmp, axis=1, keepdims=True)
        eqg = (bi == lax.broadcasted_iota(jnp.int32, (_RBLK, ng), 1)).astype(
            jnp.float32)
        k_i = jnp.dot(eqg, k_ref[...], preferred_element_type=jnp.float32)
        gate_ref[...] = jnp.where(rank < k_i, jnp.tanh(si), 0.0)

    return pl.pallas_call(
        body,
        grid=(grid,),
        in_specs=[
            pl.BlockSpec((_RBLK, 1), lambda i: (i, 0)),
            pl.BlockSpec((_RBLK, 1), lambda i: (i, 0)),
            pl.BlockSpec((_RBLK, 1), lambda i: (i, 0)),
            pl.BlockSpec((1, 1), lambda i: (0, 0)),
            pl.BlockSpec((1, n), lambda i: (0, 0)),
            pl.BlockSpec((1, n), lambda i: (0, 0)),
            pl.BlockSpec((1, n), lambda i: (0, 0)),
            pl.BlockSpec((_RBLK, 1), lambda i: (i, 0)),
            pl.BlockSpec((1, n), lambda i: (0, 0)),
            pl.BlockSpec((ng, 1), lambda i: (0, 0)),
        ],
        out_specs=pl.BlockSpec((_RBLK, 1), lambda i: (i, 0)),
        out_shape=jax.ShapeDtypeStruct((n, 1), jnp.float32),
    )(sa0, sa1, sroot, brel, sa0r, sa1r, srootr, bat_c, bat_r, karr)


def _tc_pool_linear(h, gate, bat_c, kf, w_lin, b_lin):
    n, hid = h.shape
    ng = kf.shape[0]
    grid = n // _BLK

    def body(h_ref, g_ref, b_ref, kf_ref, w_ref, bl_ref, out_ref, acc):
        j = pl.program_id(0)

        @pl.when(j == 0)
        def _():
            acc[...] = jnp.zeros((ng, hid), jnp.float32)

        hp = h_ref[...] * g_ref[...]
        eq = (b_ref[...] == lax.broadcasted_iota(jnp.int32, (_BLK, ng), 1)
              ).astype(jnp.float32)
        acc[...] += lax.dot_general(eq, hp, (((0,), (0,)), ((), ())),
                                    preferred_element_type=jnp.float32)

        @pl.when(j == grid - 1)
        def _():
            pooled = acc[...] / kf_ref[...]
            out_ref[...] = jnp.dot(pooled, w_ref[...],
                                   preferred_element_type=jnp.float32) + bl_ref[...]

    return pl.pallas_call(
        body,
        grid=(grid,),
        in_specs=[
            pl.BlockSpec((_BLK, hid), lambda i: (i, 0)),
            pl.BlockSpec((_BLK, 1), lambda i: (i, 0)),
            pl.BlockSpec((_BLK, 1), lambda i: (i, 0)),
            pl.BlockSpec((ng, 1), lambda i: (0, 0)),
            pl.BlockSpec((hid, hid), lambda i: (0, 0)),
            pl.BlockSpec((1, hid), lambda i: (0, 0)),
        ],
        out_specs=pl.BlockSpec((ng, hid), lambda i: (0, 0)),
        out_shape=jax.ShapeDtypeStruct((ng, hid), jnp.float32),
        scratch_shapes=[pltpu.VMEM((ng, hid), jnp.float32)],
    )(h, gate, bat_c, kf, w_lin, b_lin)


def kernel(x, edge_index, batch, W_gcn, b_gcn, w_rel, b_rel, w_root, W_lin, b_lin):
    n = x.shape[0]
    hid = W_gcn.shape[1]
    ng = 64
    ratio = 0.5
    e = edge_index.shape[1]
    nw = _NC * _NS
    unit = 128 * _NS
    npad = -(-n // unit) * unit
    nch = 8 * (-(-e // (nw * _CH * 8)))
    epad = nw * nch * _CH
    sent = jnp.full((epad - e,), n, jnp.int32)
    row2d = jnp.concatenate([edge_index[0], sent]).reshape(-1, _CH)
    col2d = jnp.concatenate([edge_index[1], sent]).reshape(-1, _CH)
    zeros_pad = jnp.zeros((npad,), jnp.float32)

    deg_p = _sc_hist(col2d, zeros_pad)
    d0 = deg_p[0, :n].reshape(n, 1)
    d1 = deg_p[1, :n].reshape(n, 1)

    xw = _tc_matmul(x, W_gcn)
    xws, dinv = _tc_scale(xw, d0, d1)
    xws_pad = jnp.pad(xws, ((0, npad - n), (0, 0)))

    agg_p = _sc_row_scatter(xws_pad, row2d, col2d)

    w2 = jnp.concatenate([w_rel, w_root], axis=1)
    h, s12 = _tc_finish_conv(agg_p[0, :n], agg_p[1, :n], xws, dinv,
                             b_gcn.reshape(1, hid), w2)
    s1_pad = jnp.pad(s12[:, 0], (0, npad - n))
    sroot = s12[:, 1:2]

    sagg_p = _sc_scalar_scatter(s1_pad, row2d, col2d, zeros_pad)

    karr, kf = _tc_counts(batch.reshape(1, n), ng, ratio)

    gate = _tc_rank_gate(
        sagg_p[0, :n].reshape(n, 1), sagg_p[1, :n].reshape(n, 1), sroot,
        b_rel.reshape(1, 1),
        sagg_p[0, :n].reshape(1, n), sagg_p[1, :n].reshape(1, n),
        s12[:, 1].reshape(1, n),
        batch.reshape(n, 1), batch.reshape(1, n), karr)

    return _tc_pool_linear(h, gate, batch.reshape(n, 1), kf, W_lin,
                           b_lin.reshape(1, hid))


# speedup vs baseline: 1.0091x; 1.0091x over previous
import functools

import jax
import jax.numpy as jnp
from jax import lax
from jax.experimental import pallas as pl
from jax.experimental.pallas import tpu as pltpu
from jax.experimental.pallas import tpu_sc as plsc

_NC = 2
_NS = 16
_CH = 128


def _sc_mesh():
    return plsc.VectorSubcoreMesh(core_axis_name="c", subcore_axis_name="s")


def _sc_hist(col2d, zeros_pad):
    nch_all, _ = col2d.shape
    npad = zeros_pad.shape[0]
    nw = _NC * _NS
    nch = nch_all // nw
    rpt = npad // _NS

    @functools.partial(
        pl.kernel,
        out_type=jax.ShapeDtypeStruct((_NC, npad), jnp.float32),
        mesh=_sc_mesh(),
        scratch_types=[
            pltpu.VMEM_SHARED((npad,), jnp.float32),
            pltpu.VMEM((nch, _CH), jnp.int32),
            pltpu.VMEM((_CH,), jnp.float32),
        ],
    )
    def k(col_hbm, z_hbm, out_hbm, acc, ic_all, ones_v):
        cid = lax.axis_index("c")
        sid = lax.axis_index("s")
        wid = sid * _NC + cid
        pltpu.sync_copy(z_hbm.at[pl.ds(sid * rpt, rpt)],
                        acc.at[pl.ds(sid * rpt, rpt)])
        pltpu.sync_copy(col_hbm.at[pl.ds(wid * nch, nch)], ic_all)
        for j in range(_CH // 16):
            ones_v[pl.ds(j * 16, 16)] = jnp.ones((16,), jnp.float32)
        plsc.subcore_barrier()

        def body(ch, carry):
            pltpu.sync_copy(ones_v, acc.at[ic_all.at[ch]], add=True)
            return carry

        lax.fori_loop(0, nch, body, 0)
        plsc.subcore_barrier()
        pltpu.sync_copy(acc.at[pl.ds(sid * rpt, rpt)],
                        out_hbm.at[cid].at[pl.ds(sid * rpt, rpt)])

    return k(col2d, zeros_pad)


def _sc_row_scatter(xws, row2d, col2d):
    n, d = xws.shape
    nch_all, _ = row2d.shape
    nw = _NC * _NS
    nch = nch_all // nw
    rpt = n // _NS

    @functools.partial(
        pl.kernel,
        out_type=jax.ShapeDtypeStruct((_NC, n, d), jnp.float32),
        mesh=_sc_mesh(),
        scratch_types=[
            pltpu.VMEM_SHARED((n, d), jnp.float32),
            pltpu.VMEM((nch, _CH), jnp.int32),
            pltpu.VMEM((_CH,), jnp.int32),
            pltpu.VMEM((_CH,), jnp.int32),
            pltpu.VMEM((_CH, d), jnp.float32),
            pltpu.VMEM((_CH, d), jnp.float32),
            pltpu.SemaphoreType.DMA,
            pltpu.SemaphoreType.DMA,
        ],
    )
    def k(xws_hbm, row_hbm, col_hbm, out_hbm, acc, ir_all, ic0, ic1,
          rows0, rows1, sem0, sem1):
        cid = lax.axis_index("c")
        sid = lax.axis_index("s")
        wid = sid * _NC + cid
        pltpu.sync_copy(xws_hbm.at[pl.ds(sid * rpt, rpt)],
                        acc.at[pl.ds(sid * rpt, rpt)])
        pltpu.sync_copy(row_hbm.at[pl.ds(wid * nch, nch)], ir_all)
        plsc.subcore_barrier()

        pltpu.async_copy(xws_hbm.at[ir_all.at[0]], rows0, sem0)
        pltpu.sync_copy(col_hbm.at[wid * nch], ic0)

        def body(it, carry):
            ch0 = it * 2
            ch1 = ch0 + 1
            pltpu.async_copy(xws_hbm.at[ir_all.at[ch1]], rows1, sem1)
            pltpu.sync_copy(col_hbm.at[wid * nch + ch1], ic1)
            pltpu.make_async_copy(xws_hbm.at[ir_all.at[ch0]], rows0,
                                  sem0).wait()
            pltpu.sync_copy(rows0, acc.at[ic0], add=True)

            @pl.when(ch0 + 2 < nch)
            def _():
                pltpu.async_copy(xws_hbm.at[ir_all.at[ch0 + 2]], rows0, sem0)
                pltpu.sync_copy(col_hbm.at[wid * nch + ch0 + 2], ic0)

            pltpu.make_async_copy(xws_hbm.at[ir_all.at[ch1]], rows1,
                                  sem1).wait()
            pltpu.sync_copy(rows1, acc.at[ic1], add=True)
            return carry

        lax.fori_loop(0, nch // 2, body, 0)
        plsc.subcore_barrier()
        pltpu.sync_copy(acc.at[pl.ds(sid * rpt, rpt)],
                        out_hbm.at[cid].at[pl.ds(sid * rpt, rpt)])

    return k(xws, row2d, col2d)


def _sc_scalar_scatter(s1, row2d, col2d, zeros_pad):
    npad = zeros_pad.shape[0]
    nch_all, _ = row2d.shape
    nw = _NC * _NS
    nch = nch_all // nw
    rpt = npad // _NS

    @functools.partial(
        pl.kernel,
        out_type=jax.ShapeDtypeStruct((_NC, npad), jnp.float32),
        mesh=_sc_mesh(),
        scratch_types=[
            pltpu.VMEM_SHARED((npad,), jnp.float32),
            pltpu.VMEM((nch, _CH), jnp.int32),
            pltpu.VMEM((nch, _CH), jnp.int32),
            pltpu.VMEM((_CH,), jnp.float32),
            pltpu.VMEM((_CH,), jnp.float32),
            pltpu.SemaphoreType.DMA,
            pltpu.SemaphoreType.DMA,
        ],
    )
    def k(s1_hbm, row_hbm, col_hbm, z_hbm, out_hbm, acc, ir_all, ic_all,
          val0, val1, sem0, sem1):
        cid = lax.axis_index("c")
        sid = lax.axis_index("s")
        wid = sid * _NC + cid
        pltpu.sync_copy(z_hbm.at[pl.ds(sid * rpt, rpt)],
                        acc.at[pl.ds(sid * rpt, rpt)])
        pltpu.sync_copy(row_hbm.at[pl.ds(wid * nch, nch)], ir_all)
        pltpu.sync_copy(col_hbm.at[pl.ds(wid * nch, nch)], ic_all)
        plsc.subcore_barrier()

        pltpu.async_copy(s1_hbm.at[ir_all.at[0]], val0, sem0)

        def body(it, carry):
            ch0 = it * 2
            ch1 = ch0 + 1
            pltpu.async_copy(s1_hbm.at[ir_all.at[ch1]], val1, sem1)
            pltpu.make_async_copy(s1_hbm.at[ir_all.at[ch0]], val0,
                                  sem0).wait()
            pltpu.sync_copy(val0, acc.at[ic_all.at[ch0]], add=True)

            @pl.when(ch0 + 2 < nch)
            def _():
                pltpu.async_copy(s1_hbm.at[ir_all.at[ch0 + 2]], val0, sem0)

            pltpu.make_async_copy(s1_hbm.at[ir_all.at[ch1]], val1,
                                  sem1).wait()
            pltpu.sync_copy(val1, acc.at[ic_all.at[ch1]], add=True)
            return carry

        lax.fori_loop(0, nch // 2, body, 0)
        plsc.subcore_barrier()
        pltpu.sync_copy(acc.at[pl.ds(sid * rpt, rpt)],
                        out_hbm.at[cid].at[pl.ds(sid * rpt, rpt)])

    return k(s1, row2d, col2d, zeros_pad)


_BLK = 400


def _tc_matmul(x, w):
    n, din = x.shape
    hid = w.shape[1]
    grid = n // _BLK

    def body(x_ref, w_ref, xw_ref):
        xw_ref[...] = jnp.dot(x_ref[...], w_ref[...],
                              preferred_element_type=jnp.float32)

    return pl.pallas_call(
        body,
        grid=(grid,),
        in_specs=[
            pl.BlockSpec((_BLK, din), lambda i: (i, 0)),
            pl.BlockSpec((din, hid), lambda i: (0, 0)),
        ],
        out_specs=pl.BlockSpec((_BLK, hid), lambda i: (i, 0)),
        out_shape=jax.ShapeDtypeStruct((n, hid), jnp.float32),
    )(x, w)


def _tc_scale(xw, d0, d1):
    n, hid = xw.shape
    grid = n // _BLK

    def body(xw_ref, d0_ref, d1_ref, xws_ref, dinv_ref):
        dinv = lax.rsqrt(d0_ref[...] + d1_ref[...] + 1.0)
        xws_ref[...] = xw_ref[...] * dinv
        dinv_ref[...] = dinv

    return pl.pallas_call(
        body,
        grid=(grid,),
        in_specs=[
            pl.BlockSpec((_BLK, hid), lambda i: (i, 0)),
            pl.BlockSpec((_BLK, 1), lambda i: (i, 0)),
            pl.BlockSpec((_BLK, 1), lambda i: (i, 0)),
        ],
        out_specs=[
            pl.BlockSpec((_BLK, hid), lambda i: (i, 0)),
            pl.BlockSpec((_BLK, 1), lambda i: (i, 0)),
        ],
        out_shape=[
            jax.ShapeDtypeStruct((n, hid), jnp.float32),
            jax.ShapeDtypeStruct((n, 1), jnp.float32),
        ],
    )(xw, d0, d1)


def _tc_finish_conv(p0, p1, xws, dinv, b, w2):
    n, hid = p0.shape
    grid = n // _BLK

    def body(p0_ref, p1_ref, xws_ref, dinv_ref, b_ref, w2_ref, h_ref, s_ref):
        agg = p0_ref[...] + p1_ref[...] - xws_ref[...]
        h = jnp.maximum(dinv_ref[...] * agg + b_ref[...], 0.0)
        h_ref[...] = h
        s_ref[...] = jnp.dot(h, w2_ref[...], preferred_element_type=jnp.float32)

    return pl.pallas_call(
        body,
        grid=(grid,),
        in_specs=[
            pl.BlockSpec((_BLK, hid), lambda i: (i, 0)),
            pl.BlockSpec((_BLK, hid), lambda i: (i, 0)),
            pl.BlockSpec((_BLK, hid), lambda i: (i, 0)),
            pl.BlockSpec((_BLK, 1), lambda i: (i, 0)),
            pl.BlockSpec((1, hid), lambda i: (0, 0)),
            pl.BlockSpec((hid, 2), lambda i: (0, 0)),
        ],
        out_specs=[
            pl.BlockSpec((_BLK, hid), lambda i: (i, 0)),
            pl.BlockSpec((_BLK, 2), lambda i: (i, 0)),
        ],
        out_shape=[
            jax.ShapeDtypeStruct((n, hid), jnp.float32),
            jax.ShapeDtypeStruct((n, 2), jnp.float32),
        ],
    )(p0, p1, xws, dinv, b, w2)


def _tc_counts(batch_row, num_graphs, ratio):
    n = batch_row.shape[1]

    def body(b_ref, k_ref, kf_ref):
        g = lax.broadcasted_iota(jnp.int32, (num_graphs, n), 0)
        eq = (b_ref[...] == g).astype(jnp.float32)
        counts = jnp.sum(eq, axis=1, keepdims=True)
        kk = jnp.ceil(ratio * counts)
        k_ref[...] = kk
        kf_ref[...] = jnp.maximum(kk, 1.0)

    return pl.pallas_call(
        body,
        out_shape=[
            jax.ShapeDtypeStruct((num_graphs, 1), jnp.float32),
            jax.ShapeDtypeStruct((num_graphs, 1), jnp.float32),
        ],
    )(batch_row)


_RBLK = 200


def _tc_rank_gate(sa0, sa1, sroot, brel, sa0r, sa1r, srootr, bat_c, bat_r, karr):
    n = sa0.shape[0]
    ng = karr.shape[0]
    grid = n // _RBLK

    def body(sa0_ref, sa1_ref, sroot_ref, brel_ref, sa0r_ref, sa1r_ref,
             srootr_ref, batc_ref, batr_ref, k_ref, gate_ref):
        i = pl.program_id(0)
        si = sa0_ref[...] + sa1_ref[...] + brel_ref[...] + sroot_ref[...]
        sj = sa0r_ref[...] + sa1r_ref[...] + brel_ref[...] + srootr_ref[...]
        bi = batc_ref[...]
        bj = batr_ref[...]
        pos_i = i * _RBLK + lax.broadcasted_iota(jnp.int32, (_RBLK, 1), 0)
        pos_j = lax.broadcasted_iota(jnp.int32, (1, n), 1)
        beats = (sj > si) | ((sj == si) & (pos_j < pos_i))
        cmp = ((bj == bi) & beats).astype(jnp.float32)
        rank = jnp.sum(cmp, axis=1, keepdims=True)
        eqg = (bi == lax.broadcasted_iota(jnp.int32, (_RBLK, ng), 1)).astype(
            jnp.float32)
        k_i = jnp.dot(eqg, k_ref[...], preferred_element_type=jnp.float32)
        gate_ref[...] = jnp.where(rank < k_i, jnp.tanh(si), 0.0)

    return pl.pallas_call(
        body,
        grid=(grid,),
        in_specs=[
            pl.BlockSpec((_RBLK, 1), lambda i: (i, 0)),
            pl.BlockSpec((_RBLK, 1), lambda i: (i, 0)),
            pl.BlockSpec((_RBLK, 1), lambda i: (i, 0)),
            pl.BlockSpec((1, 1), lambda i: (0, 0)),
            pl.BlockSpec((1, n), lambda i: (0, 0)),
            pl.BlockSpec((1, n), lambda i: (0, 0)),
            pl.BlockSpec((1, n), lambda i: (0, 0)),
            pl.BlockSpec((_RBLK, 1), lambda i: (i, 0)),
            pl.BlockSpec((1, n), lambda i: (0, 0)),
            pl.BlockSpec((ng, 1), lambda i: (0, 0)),
        ],
        out_specs=pl.BlockSpec((_RBLK, 1), lambda i: (i, 0)),
        out_shape=jax.ShapeDtypeStruct((n, 1), jnp.float32),
    )(sa0, sa1, sroot, brel, sa0r, sa1r, srootr, bat_c, bat_r, karr)


def _tc_pool_linear(h, gate, bat_c, kf, w_lin, b_lin):
    n, hid = h.shape
    ng = kf.shape[0]
    grid = n // _BLK

    def body(h_ref, g_ref, b_ref, kf_ref, w_ref, bl_ref, out_ref, acc):
        j = pl.program_id(0)

        @pl.when(j == 0)
        def _():
            acc[...] = jnp.zeros((ng, hid), jnp.float32)

        hp = h_ref[...] * g_ref[...]
        eq = (b_ref[...] == lax.broadcasted_iota(jnp.int32, (_BLK, ng), 1)
              ).astype(jnp.float32)
        acc[...] += lax.dot_general(eq, hp, (((0,), (0,)), ((), ())),
                                    preferred_element_type=jnp.float32)

        @pl.when(j == grid - 1)
        def _():
            pooled = acc[...] / kf_ref[...]
            out_ref[...] = jnp.dot(pooled, w_ref[...],
                                   preferred_element_type=jnp.float32) + bl_ref[...]

    return pl.pallas_call(
        body,
        grid=(grid,),
        in_specs=[
            pl.BlockSpec((_BLK, hid), lambda i: (i, 0)),
            pl.BlockSpec((_BLK, 1), lambda i: (i, 0)),
            pl.BlockSpec((_BLK, 1), lambda i: (i, 0)),
            pl.BlockSpec((ng, 1), lambda i: (0, 0)),
            pl.BlockSpec((hid, hid), lambda i: (0, 0)),
            pl.BlockSpec((1, hid), lambda i: (0, 0)),
        ],
        out_specs=pl.BlockSpec((ng, hid), lambda i: (0, 0)),
        out_shape=jax.ShapeDtypeStruct((ng, hid), jnp.float32),
        scratch_shapes=[pltpu.VMEM((ng, hid), jnp.float32)],
    )(h, gate, bat_c, kf, w_lin, b_lin)


def kernel(x, edge_index, batch, W_gcn, b_gcn, w_rel, b_rel, w_root, W_lin, b_lin):
    n = x.shape[0]
    hid = W_gcn.shape[1]
    ng = 64
    ratio = 0.5
    e = edge_index.shape[1]
    nw = _NC * _NS
    unit = 128 * _NS
    npad = -(-n // unit) * unit
    nch = 8 * (-(-e // (nw * _CH * 8)))
    epad = nw * nch * _CH
    sent = jnp.full((epad - e,), n, jnp.int32)
    row2d = jnp.concatenate([edge_index[0], sent]).reshape(-1, _CH)
    col2d = jnp.concatenate([edge_index[1], sent]).reshape(-1, _CH)
    zeros_pad = jnp.zeros((npad,), jnp.float32)

    deg_p = _sc_hist(col2d, zeros_pad)
    d0 = deg_p[0, :n].reshape(n, 1)
    d1 = deg_p[1, :n].reshape(n, 1)

    xw = _tc_matmul(x, W_gcn)
    xws, dinv = _tc_scale(xw, d0, d1)
    xws_pad = jnp.pad(xws, ((0, npad - n), (0, 0)))

    agg_p = _sc_row_scatter(xws_pad, row2d, col2d)

    w2 = jnp.concatenate([w_rel, w_root], axis=1)
    h, s12 = _tc_finish_conv(agg_p[0, :n], agg_p[1, :n], xws, dinv,
                             b_gcn.reshape(1, hid), w2)
    s1_pad = jnp.pad(s12[:, 0], (0, npad - n))
    sroot = s12[:, 1:2]

    sagg_p = _sc_scalar_scatter(s1_pad, row2d, col2d, zeros_pad)

    karr, kf = _tc_counts(batch.reshape(1, n), ng, ratio)

    gate = _tc_rank_gate(
        sagg_p[0, :n].reshape(n, 1), sagg_p[1, :n].reshape(n, 1), sroot,
        b_rel.reshape(1, 1),
        sagg_p[0, :n].reshape(1, n), sagg_p[1, :n].reshape(1, n),
        s12[:, 1].reshape(1, n),
        batch.reshape(n, 1), batch.reshape(1, n), karr)

    return _tc_pool_linear(h, gate, batch.reshape(n, 1), kf, W_lin,
                           b_lin.reshape(1, hid))
